# baseline (speedup 1.0000x reference)
import functools

import jax
import jax.numpy as jnp
from jax import lax
from jax.experimental import pallas as pl
from jax.experimental.pallas import tpu as pltpu
from jax.experimental.pallas import tpu_sc as plsc

N = 10000
E = 320000
D = 128

NS = 16
EB = 112
NB = 180
TRIP = NB // 3
PEPT = NB * EB
EPAD = NS * PEPT
RPT = 624
ZR = 16


def _make_phase(h_hbm, out_hbm, acc, EBUF, VB, SB, GB, zbuf, ES, GS, SS, s):
    def run(rc_hbm, v_hbm, oi):
        def fire_edges(b, g):
            off = (s * NB + g) * (2 * EB)
            pltpu.async_copy(rc_hbm.at[pl.ds(off, 2 * EB)], EBUF[b], ES[b])
            voff = s * PEPT + g * EB
            pltpu.async_copy(v_hbm.at[pl.ds(voff, EB)], VB[b], ES[b])

        def wait_edges(b):
            pltpu.make_async_copy(rc_hbm.at[pl.ds(0, 2 * EB)],
                                  EBUF[b], ES[b]).wait()
            pltpu.make_async_copy(v_hbm.at[pl.ds(0, EB)],
                                  VB[b], ES[b]).wait()

        def start_gather(b):
            pltpu.async_copy(h_hbm.at[EBUF[b].at[pl.ds(EB, EB)]],
                             GB[b], GS[b])

        def wait_gather(b):
            pltpu.make_async_copy(h_hbm.at[EBUF[b].at[pl.ds(EB, EB)]],
                                  GB[b], GS[b]).wait()

        def fire_scatter(b):
            pltpu.async_copy(GB[b], acc.at[SB[b]], SS[b], add=True)

        def wait_scatter(b):
            pltpu.make_async_copy(GB[b], acc.at[SB[b]], SS[b]).wait()

        def compute(b):
            for k in range(EB // 16):
                sl = pl.ds(k * 16, 16)
                SB[b][sl] = EBUF[b][sl]

            gbuf = GB[b]

            def grp(gg, _):
                vv = VB[b][pl.ds(gg * 16, 16)]
                for e16 in range(16):
                    e = gg * 16 + e16
                    v = vv[e16]
                    for dd in range(8):
                        sl = pl.ds(dd * 16, 16)
                        gbuf[e, sl] = gbuf[e, sl] * v
                return 0
            lax.fori_loop(0, EB // 16, grp, 0)

        fire_edges(0, 0)
        fire_edges(1, 1)
        fire_edges(2, 2)

        def zrow(i, _):
            for dd in range(8):
                zbuf[i, pl.ds(dd * 16, 16)] = jnp.zeros((16,), jnp.float32)
            return 0
        lax.fori_loop(0, ZR, zrow, 0)

        def zcopy(j, _):
            pltpu.sync_copy(zbuf, acc.at[pl.ds(s * RPT + j * ZR, ZR)])
            return 0
        lax.fori_loop(0, RPT // ZR, zcopy, 0)

        @pl.when(s == NS - 1)
        def _():
            pltpu.sync_copy(zbuf, acc.at[pl.ds(NS * RPT, 16)])

        wait_edges(0)
        start_gather(0)
        plsc.subcore_barrier()

        def trip(q, _):
            more = q < TRIP - 1
            for k in range(3):
                b = k
                b1 = (k + 1) % 3
                if k < 2:
                    wait_edges(b1)

                    @pl.when(q > 0)
                    def _():
                        wait_scatter(b1)
                    start_gather(b1)
                else:
                    @pl.when(more)
                    def _():
                        wait_edges(b1)
                        wait_scatter(b1)
                        start_gather(b1)
                wait_gather(b)
                compute(b)
                fire_scatter(b)

                @pl.when(more)
                def _():
                    fire_edges(b, 3 * q + k + 3)
            return 0

        lax.fori_loop(0, TRIP, trip, 0)
        wait_scatter(0)
        wait_scatter(1)
        wait_scatter(2)

        plsc.subcore_barrier()

        pltpu.sync_copy(acc.at[pl.ds(s * RPT, RPT)],
                        out_hbm.at[oi, pl.ds(s * RPT, RPT)])

        @pl.when(s == NS - 1)
        def _():
            pltpu.sync_copy(acc.at[pl.ds(NS * RPT, 16)],
                            out_hbm.at[oi, pl.ds(NS * RPT, 16)])

    return run


def _scratch_args(args):
    ebufs, vbs, sbs, gbs = args[0:3], args[3:6], args[6:9], args[9:12]
    zbuf = args[12]
    ess, gss, sss = args[13:16], args[16:19], args[19:22]
    return ebufs, vbs, sbs, gbs, zbuf, ess, gss, sss


def _spmm2_body(h_hbm, rc0_hbm, v0_hbm, rc1_hbm, v1_hbm, out_hbm,
                acc, *scr):
    c = lax.axis_index("c")
    s = lax.axis_index("s")
    EBUF, VB, SB, GB, zbuf, ES, GS, SS = _scratch_args(scr)
    run = _make_phase(h_hbm, out_hbm, acc, EBUF, VB, SB, GB, zbuf,
                      ES, GS, SS, s)

    @pl.when(c == 0)
    def _():
        run(rc0_hbm, v0_hbm, 0)

    @pl.when(c == 1)
    def _():
        run(rc1_hbm, v1_hbm, 1)


def _spmm4_body(h_hbm, rc0_hbm, v0_hbm, rc1_hbm, v1_hbm,
                rc2_hbm, v2_hbm, rc3_hbm, v3_hbm, out_hbm,
                acc, *scr):
    c = lax.axis_index("c")
    s = lax.axis_index("s")
    EBUF, VB, SB, GB, zbuf, ES, GS, SS = _scratch_args(scr)
    run = _make_phase(h_hbm, out_hbm, acc, EBUF, VB, SB, GB, zbuf,
                      ES, GS, SS, s)

    @pl.when(c == 0)
    def _():
        run(rc0_hbm, v0_hbm, 0)
        run(rc2_hbm, v2_hbm, 2)

    @pl.when(c == 1)
    def _():
        run(rc1_hbm, v1_hbm, 1)
        run(rc3_hbm, v3_hbm, 3)


_SCRATCH = (
    [pltpu.VMEM_SHARED((N, D), jnp.float32)]
    + [pltpu.VMEM((2 * EB,), jnp.int32)] * 3
    + [pltpu.VMEM((EB,), jnp.float32)] * 3
    + [pltpu.VMEM((EB,), jnp.int32)] * 3
    + [pltpu.VMEM((EB, D), jnp.float32)] * 3
    + [pltpu.VMEM((ZR, D), jnp.float32)]
    + [pltpu.SemaphoreType.DMA] * 9
)

_spmm2 = functools.partial(
    pl.kernel,
    out_type=jax.ShapeDtypeStruct((2, N, D), jnp.float32),
    mesh=plsc.VectorSubcoreMesh(core_axis_name="c", subcore_axis_name="s"),
    scratch_types=list(_SCRATCH),
)(_spmm2_body)

_spmm4 = functools.partial(
    pl.kernel,
    out_type=jax.ShapeDtypeStruct((4, N, D), jnp.float32),
    mesh=plsc.VectorSubcoreMesh(core_axis_name="c", subcore_axis_name="s"),
    scratch_types=list(_SCRATCH),
)(_spmm4_body)


_BLK = 1000
_GRID = N // _BLK

_row_spec = pl.BlockSpec((_BLK, D), lambda i: (i, 0))
_smem_spec = pl.BlockSpec(memory_space=pltpu.SMEM)


def _affine_body(x_ref, wt_ref, b_ref, o_ref):
    o_ref[...] = jnp.dot(x_ref[...], wt_ref[...],
                         preferred_element_type=jnp.float32) + b_ref[...]


def _affine(x, wt, b2d):
    return pl.pallas_call(
        _affine_body,
        grid=(_GRID,),
        in_specs=[_row_spec,
                  pl.BlockSpec((D, D), lambda i: (0, 0)),
                  pl.BlockSpec((1, D), lambda i: (0, 0))],
        out_specs=_row_spec,
        out_shape=jax.ShapeDtypeStruct((N, D), jnp.float32),
    )(x, wt, b2d)


def _combine1_body(c_ref, y0, y1, y2, y3, s1o, r1o, oro):
    a, b, cc, d = y0[...], y1[...], y2[...], y3[...]
    s1o[...] = c_ref[0] * a + c_ref[1] * b + c_ref[2] * cc
    r1o[...] = c_ref[3] * a + c_ref[4] * b + c_ref[5] * cc + c_ref[6] * d
    oro[...] = c_ref[7] * a + c_ref[8] * b + c_ref[9] * d


def _combine1(cvec, y0, y1, y2, y3):
    nd = jax.ShapeDtypeStruct((N, D), jnp.float32)
    return pl.pallas_call(
        _combine1_body,
        grid=(_GRID,),
        in_specs=[_smem_spec] + [_row_spec] * 4,
        out_specs=[_row_spec] * 3,
        out_shape=[nd, nd, nd],
    )(cvec, y0, y1, y2, y3)


def _combine2_body(c_ref, z0, z1, z2, z3, r1, ora, s2o, orto):
    a, b, cc, d = z0[...], z1[...], z2[...], z3[...]
    s2o[...] = c_ref[0] * a + c_ref[1] * b + c_ref[2] * cc + r1[...]
    orto[...] = ora[...] + c_ref[3] * a + c_ref[4] * b + c_ref[5] * d


def _combine2(cvec, z0, z1, z2, z3, r1, ora):
    nd = jax.ShapeDtypeStruct((N, D), jnp.float32)
    return pl.pallas_call(
        _combine2_body,
        grid=(_GRID,),
        in_specs=[_smem_spec] + [_row_spec] * 6,
        out_specs=[_row_spec] * 2,
        out_shape=[nd, nd],
    )(cvec, z0, z1, z2, z3, r1, ora)


def _final_body(c_ref, u0, u1, ort, o_ref):
    h = c_ref[0] * u0[...] + c_ref[1] * u1[...] + ort[...]
    mu = jnp.mean(h, axis=-1, keepdims=True)
    var = jnp.mean((h - mu) ** 2, axis=-1, keepdims=True)
    t = (h - mu) / jnp.sqrt(var + 1e-5)
    o_ref[...] = t * 0.5 * (1.0 + lax.erf(t * 0.7071067811865476))


def _final(cvec, u0, u1, ort):
    return pl.pallas_call(
        _final_body,
        grid=(_GRID,),
        in_specs=[_smem_spec] + [_row_spec] * 3,
        out_specs=_row_spec,
        out_shape=jax.ShapeDtypeStruct((N, D), jnp.float32),
    )(cvec, u0, u1, ort)


def kernel(x, adj_indices, adj_values, ws_seq_0, ws_seq_1, ws_res_0,
           ws_res_1, W_affine, b_affine):
    h = _affine(x, W_affine.T, b_affine.reshape(1, D))

    TB = NS * NB
    ipad = jnp.zeros((EPAD - E,), jnp.int32)
    fpad = jnp.zeros((EPAD - E,), jnp.float32)

    def pack_edges(i):
        rp = jnp.concatenate([adj_indices[i, 0], ipad]).reshape(TB, EB)
        cp = jnp.concatenate([adj_indices[i, 1], ipad]).reshape(TB, EB)
        return jnp.stack([rp, cp], axis=1).reshape(-1)

    rc = [pack_edges(i) for i in range(4)]
    v = [jnp.concatenate([adj_values[i], fpad]) for i in range(4)]

    y = _spmm4(h, rc[0], v[0], rc[1], v[1], rc[2], v[2], rc[3], v[3])
    c1 = jnp.concatenate([ws_seq_0[0] / 3.0, ws_res_0[0] / 4.0,
                          ws_res_1[0] / 3.0])
    s1, res1, ora = _combine1(c1, y[0], y[1], y[2], y[3])

    z = _spmm4(s1, rc[0], v[0], rc[1], v[1], rc[2], v[2], rc[3], v[3])
    c2 = jnp.concatenate([ws_seq_0[1] / 3.0, ws_res_1[1] / 3.0])
    s2, ort = _combine2(c2, z[0], z[1], z[2], z[3], res1, ora)

    u = _spmm2(s2, rc[0], v[0], rc[1], v[1])
    return _final(ws_seq_1 / 2.0, u[0], u[1], ort)
